# Initial kernel scaffold; baseline (speedup 1.0000x reference)
import jax
import jax.numpy as jnp
from jax.experimental import pallas as pl


def kernel(hidden_states, weight):
    raise NotImplementedError("write your pallas kernel here")


# speedup vs baseline: 1.4241x; 1.4241x over previous
import functools

import jax
import jax.numpy as jnp
from jax.experimental import pallas as pl
import jax.experimental.pallas.tpu as pltpu

N_EXP = 256
N_GRP = 8
GRP = 32
TOPK_GRP = 4
TOPK = 8
HID = 2048
TOK = 8192
ALPHA = 0.001
BLK = 256
NEG = -3.0e38


def _gate_kernel(h_ref, wt_ref, idx_ref, w_ref, aux_ref, cnt_ref, psum_ref):
    step = pl.program_id(0)
    nsteps = pl.num_programs(0)

    @pl.when(step == 0)
    def _init():
        cnt_ref[...] = jnp.zeros_like(cnt_ref)
        psum_ref[...] = jnp.zeros_like(psum_ref)

    logits = jnp.dot(h_ref[...], wt_ref[...],
                     preferred_element_type=jnp.float32)

    g3 = logits.reshape(BLK, N_GRP, GRP)
    work = g3
    m = None
    for _ in range(TOPK_GRP):
        m = jnp.max(work, axis=-1, keepdims=True)
        work = jnp.where(work >= m, NEG, work)
    cand = jnp.where(g3 >= m, g3, NEG).reshape(BLK, N_EXP)

    lane = jax.lax.broadcasted_iota(jnp.int32, (BLK, N_EXP), 1)
    work2 = cand
    vals = []
    idxs = []
    sel = jnp.zeros((BLK, N_EXP), jnp.float32)
    for _ in range(TOPK):
        mv = jnp.max(work2, axis=-1, keepdims=True)
        is_m = work2 >= mv
        ii = jnp.min(jnp.where(is_m, lane, N_EXP), axis=-1, keepdims=True)
        hit = lane == ii
        sel = sel + hit.astype(jnp.float32)
        work2 = jnp.where(hit, NEG, work2)
        vals.append(mv)
        idxs.append(ii)
    tw = jnp.concatenate(vals, axis=1)
    ti = jnp.concatenate(idxs, axis=1)

    idx_ref[...] = ti
    w_ref[...] = tw / (jnp.sum(tw, axis=-1, keepdims=True) + 1e-20)

    mx = jnp.max(logits, axis=-1, keepdims=True)
    e = jnp.exp(logits - mx)
    p = e / jnp.sum(e, axis=-1, keepdims=True)
    psum_ref[...] += jnp.sum(p, axis=0, keepdims=True)
    cnt_ref[...] += jnp.sum(sel, axis=0, keepdims=True)

    @pl.when(step == nsteps - 1)
    def _fin():
        counts = cnt_ref[...] / (TOK * jnp.float32(TOPK))
        probs = psum_ref[...] / jnp.float32(TOK)
        aux_ref[...] = (jnp.sum(counts * probs) * ALPHA).reshape(1, 1)


@functools.partial(jax.jit)
def _gate(hidden_states, wt):
    grid = TOK // BLK
    idx, w, aux = pl.pallas_call(
        _gate_kernel,
        grid=(grid,),
        in_specs=[
            pl.BlockSpec((BLK, HID), lambda i: (i, 0)),
            pl.BlockSpec((HID, N_EXP), lambda i: (0, 0)),
        ],
        out_specs=[
            pl.BlockSpec((BLK, TOPK), lambda i: (i, 0)),
            pl.BlockSpec((BLK, TOPK), lambda i: (i, 0)),
            pl.BlockSpec((1, 1), lambda i: (0, 0)),
        ],
        out_shape=[
            jax.ShapeDtypeStruct((TOK, TOPK), jnp.int32),
            jax.ShapeDtypeStruct((TOK, TOPK), jnp.float32),
            jax.ShapeDtypeStruct((1, 1), jnp.float32),
        ],
        scratch_shapes=[
            pltpu.VMEM((1, N_EXP), jnp.float32),
            pltpu.VMEM((1, N_EXP), jnp.float32),
        ],
    )(hidden_states, wt)
    return idx, w, aux[0, 0]


def kernel(hidden_states, weight):
    return _gate(hidden_states, weight.T)


# speedup vs baseline: 2.0058x; 1.4085x over previous
import functools

import jax
import jax.numpy as jnp
import numpy as np
from jax.experimental import pallas as pl
import jax.experimental.pallas.tpu as pltpu

N_EXP = 256
N_GRP = 8
GRP = 32
TOPK_GRP = 4
TOPK = 8
HID = 2048
TOK = 8192
ALPHA = 0.001
BLK = 256
NEG = -3.0e38


def _gmax(a0, a1):
    c = jnp.maximum(a0, a1)
    for sh in (8, 16, 32, 64):
        c = jnp.maximum(c, pltpu.roll(c, sh, 1))
    return c


def _gate_kernel(h_ref, wt_ref, idx_ref, w_ref, aux_ref, cnt_ref, psum_ref):
    step = pl.program_id(0)
    nsteps = pl.num_programs(0)

    @pl.when(step == 0)
    def _init():
        cnt_ref[...] = jnp.zeros_like(cnt_ref)
        psum_ref[...] = jnp.zeros_like(psum_ref)

    logits = jnp.dot(h_ref[...], wt_ref[...],
                     preferred_element_type=jnp.float32)
    l0 = logits[:, :128]
    l1 = logits[:, 128:]

    a0, a1 = l0, l1
    t4 = None
    for _ in range(TOPK_GRP):
        t4 = _gmax(a0, a1)
        a0 = jnp.where(a0 >= t4, NEG, a0)
        a1 = jnp.where(a1 >= t4, NEG, a1)
    c0 = jnp.where(l0 >= t4, l0, NEG)
    c1 = jnp.where(l1 >= t4, l1, NEG)

    lane0 = jax.lax.broadcasted_iota(jnp.int32, (BLK, 128), 1).astype(jnp.float32)
    lane1 = lane0 + 128.0
    w0, w1 = c0, c1
    vals = []
    idxs = []
    for _ in range(TOPK):
        mv = jnp.max(jnp.maximum(w0, w1), axis=-1, keepdims=True)
        is0 = w0 >= mv
        is1 = w1 >= mv
        il = jnp.minimum(jnp.where(is0, lane0, 1e9),
                         jnp.where(is1, lane1, 1e9))
        ii = jnp.min(il, axis=-1, keepdims=True)
        vals.append(mv)
        idxs.append(ii)
        w0 = jnp.where(is0, NEG, w0)
        w1 = jnp.where(is1, NEG, w1)
    tw = jnp.concatenate(vals, axis=1)
    tip = jnp.concatenate(idxs, axis=1).astype(jnp.int32)
    idx_ref[...] = (tip & 7) * GRP + (tip >> 3)
    w_ref[...] = tw / (jnp.sum(tw, axis=-1, keepdims=True) + 1e-20)

    e0 = jnp.exp(l0)
    e1 = jnp.exp(l1)
    den = jnp.sum(e0 + e1, axis=-1, keepdims=True)
    r = 1.0 / den
    sel0 = (w0 != c0).astype(jnp.float32)
    sel1 = (w1 != c1).astype(jnp.float32)
    psum_ref[:, :128] += jnp.sum(e0 * r, axis=0, keepdims=True)
    psum_ref[:, 128:] += jnp.sum(e1 * r, axis=0, keepdims=True)
    cnt_ref[:, :128] += jnp.sum(sel0, axis=0, keepdims=True)
    cnt_ref[:, 128:] += jnp.sum(sel1, axis=0, keepdims=True)

    @pl.when(step == nsteps - 1)
    def _fin():
        counts = cnt_ref[...] / (TOK * jnp.float32(TOPK))
        probs = psum_ref[...] / jnp.float32(TOK)
        aux_ref[...] = (jnp.sum(counts * probs) * ALPHA).reshape(1, 1)


@jax.jit
def _gate(hidden_states, wp):
    grid = TOK // BLK
    idx, w, aux = pl.pallas_call(
        _gate_kernel,
        grid=(grid,),
        in_specs=[
            pl.BlockSpec((BLK, HID), lambda i: (i, 0)),
            pl.BlockSpec((HID, N_EXP), lambda i: (0, 0)),
        ],
        out_specs=[
            pl.BlockSpec((BLK, TOPK), lambda i: (i, 0)),
            pl.BlockSpec((BLK, TOPK), lambda i: (i, 0)),
            pl.BlockSpec((1, 1), lambda i: (0, 0)),
        ],
        out_shape=[
            jax.ShapeDtypeStruct((TOK, TOPK), jnp.int32),
            jax.ShapeDtypeStruct((TOK, TOPK), jnp.float32),
            jax.ShapeDtypeStruct((1, 1), jnp.float32),
        ],
        scratch_shapes=[
            pltpu.VMEM((1, N_EXP), jnp.float32),
            pltpu.VMEM((1, N_EXP), jnp.float32),
        ],
    )(hidden_states, wp)
    return idx, w, aux[0, 0]


_PERM = np.arange(256)
_PERM = (_PERM % 8) * 32 + _PERM // 8


def kernel(hidden_states, weight):
    wp = weight[_PERM].T
    return _gate(hidden_states, wp)
